# baseline (speedup 1.0000x reference)
import dataclasses
import functools

import jax
import jax.numpy as jnp
from jax import lax
from jax.experimental import pallas as pl
from jax.experimental.pallas import tpu as pltpu
from jax.experimental.pallas import tpu_sc as plsc

_N = 10000
_E = 320000
_D = 128
_ALPHA = 0.1
_EPS = 1e-5

_NC = 2
_NS = 16
_L = 16
_NW = _NC * _NS

_CH = 80
_CB = 8
_NCH = _E // _CH
_NBT = _NCH // _CB
_BPT = 16
_BPTD = 32
_NPAD = 10240
_RPT = _NPAD // _NS


def _rsqrt16(d):
    i = plsc.bitcast(d, jnp.int32)
    i = jnp.int32(0x5F3759DF) - lax.shift_right_logical(i, 1)
    y = plsc.bitcast(i, jnp.float32)
    for _ in range(3):
        y = y * (1.5 - 0.5 * d * y * y)
    return y


def _sc_gcn_agg(x, e3, ew3):
    mesh = plsc.VectorSubcoreMesh(core_axis_name="c", subcore_axis_name="s")
    cp = pltpu.CompilerParams()
    if "needs_layout_passes" in pltpu.CompilerParams.__dataclass_fields__:
        cp = dataclasses.replace(cp, needs_layout_passes=False)
    if "use_tc_tiling_on_sc" in pltpu.CompilerParams.__dataclass_fields__:
        cp = dataclasses.replace(cp, use_tc_tiling_on_sc=False)

    @functools.partial(
        pl.kernel,
        compiler_params=cp,
        out_type=(
            jax.ShapeDtypeStruct((_NC, _NPAD, _D), jnp.float32),
            jax.ShapeDtypeStruct((_N,), jnp.float32),
        ),
        mesh=mesh,
        scratch_types=[
            pltpu.VMEM((_CB, _CH), jnp.int32),
            pltpu.VMEM((_CB, _CH), jnp.int32),
            pltpu.VMEM((_CB, _CH), jnp.float32),
            pltpu.VMEM((_N,), jnp.float32),
            pltpu.VMEM((2, _CH, _L), jnp.float32),
            pltpu.VMEM((2, _CH, _D), jnp.float32),
            pltpu.VMEM_SHARED((_NPAD, _L), jnp.float32),
            pltpu.VMEM_SHARED((_NPAD, _D), jnp.float32),
            pltpu.VMEM_SHARED((_N,), jnp.float32),
        ] + [pltpu.SemaphoreType.DMA] * 6,
    )
    def k(x_hbm, e3_hbm, ew3_hbm, agg_out, dinv_out,
          rowblk, colblk, ewblk, dinvv, valbuf, msgbuf,
          degmat, aggsh, dinvsh,
          sg0, sg1, ss0, ss1, sd0, sd1):
        semg = [sg0, sg1]
        sems = [ss0, ss1]
        semd = [sd0, sd1]
        c = lax.axis_index("c")
        s = lax.axis_index("s")
        iota16 = lax.iota(jnp.int32, _L)
        zero16i = jnp.zeros((_L,), jnp.int32)
        z16 = jnp.zeros((_L,), jnp.float32)

        with jax.named_scope("ph_zero"):
            @pl.loop(0, _CH)
            def _(e):
                valbuf[0, e, :] = z16
                valbuf[1, e, :] = z16
                for g in range(_D // _L):
                    msgbuf[0, e, pl.ds(g * _L, _L)] = z16

            for i in range(_RPT // _CH):
                base = s * _RPT + i * _CH
                pltpu.sync_copy(valbuf.at[0], degmat.at[pl.ds(base, _CH)])
                pltpu.sync_copy(msgbuf.at[0], aggsh.at[pl.ds(base, _CH)])

            plsc.subcore_barrier()

        with jax.named_scope("ph_deg"):
            @pl.loop(0, _BPTD)
            def _(jb):
                gb = s * _BPTD + jb

                @pl.when(gb < _NBT)
                def _():
                    pltpu.sync_copy(e3_hbm.at[1, pl.ds(gb * _CB, _CB)],
                                    colblk)
                    pltpu.sync_copy(ew3_hbm.at[pl.ds(gb * _CB, _CB)], ewblk)
                    hs = {}
                    for j8 in range(_CB):
                        b = j8 % 2
                        if j8 >= 2:
                            hs[b].wait()
                        for g in range(_CH // _L):
                            ew16 = ewblk[j8, pl.ds(g * _L, _L)]
                            plsc.store_scatter(
                                valbuf.at[b], [g * _L + iota16, zero16i],
                                ew16)
                        hs[b] = pltpu.async_copy(
                            valbuf.at[b], degmat.at[colblk.at[j8]],
                            semd[b], add=True)
                    hs[0].wait()
                    hs[1].wait()

            plsc.subcore_barrier()

        with jax.named_scope("ph_rsqrt"):
            for seg in range(_RPT // _CH):
                base = s * _RPT + seg * _CH
                pltpu.sync_copy(degmat.at[pl.ds(base, _CH)], valbuf.at[0])

                @pl.loop(0, _CH // _L)
                def _(g, base=base):
                    gbase = base + g * _L

                    @pl.when(gbase < _N)
                    def _():
                        r16 = g * _L + iota16
                        d16 = plsc.load_gather(valbuf.at[0], [r16, zero16i])
                        dinvv[pl.ds(gbase, _L)] = _rsqrt16(d16 + 1.0)

            @pl.when(s < _NS - 1)
            def _():
                pltpu.sync_copy(dinvv.at[pl.ds(s * _RPT, _RPT)],
                                dinvsh.at[pl.ds(s * _RPT, _RPT)])

            @pl.when(s == _NS - 1)
            def _():
                pltpu.sync_copy(dinvv.at[pl.ds(s * _RPT, _N % _RPT)],
                                dinvsh.at[pl.ds(s * _RPT, _N % _RPT)])

            plsc.subcore_barrier()
            pltpu.sync_copy(dinvsh, dinvv)

        with jax.named_scope("ph_agg"):
            w = c * _NS + s

            @pl.loop(0, _BPT)
            def _(jb):
                gb = w * _BPT + jb

                @pl.when(gb < _NBT)
                def _():
                    pltpu.sync_copy(e3_hbm.at[0, pl.ds(gb * _CB, _CB)],
                                    rowblk)
                    pltpu.sync_copy(e3_hbm.at[1, pl.ds(gb * _CB, _CB)],
                                    colblk)
                    pltpu.sync_copy(ew3_hbm.at[pl.ds(gb * _CB, _CB)], ewblk)
                    hg, hsc = {}, {}
                    hg[0] = pltpu.async_copy(
                        x_hbm.at[rowblk.at[0]], msgbuf.at[0], semg[0])
                    for j8 in range(_CB):
                        b = j8 % 2
                        hg[b].wait()
                        if j8 + 1 < _CB:
                            b1 = (j8 + 1) % 2
                            if j8 >= 1:
                                hsc[b1].wait()
                            hg[b1] = pltpu.async_copy(
                                x_hbm.at[rowblk.at[j8 + 1]], msgbuf.at[b1],
                                semg[b1])
                        for g in range(_CH // _L):
                            sl = pl.ds(g * _L, _L)
                            r16 = rowblk[j8, sl]
                            c16 = colblk[j8, sl]
                            ew16 = ewblk[j8, sl]
                            dr = plsc.load_gather(dinvv, [r16])
                            dc = plsc.load_gather(dinvv, [c16])
                            n16 = dr * ew16 * dc
                            for kk in range(_L):
                                ne = n16[kk]
                                e = g * _L + kk
                                for gg in range(_D // _L):
                                    sld = pl.ds(gg * _L, _L)
                                    msgbuf[b, e, sld] = msgbuf[b, e, sld] * ne

                        hsc[b] = pltpu.async_copy(
                            msgbuf.at[b], aggsh.at[colblk.at[j8]],
                            sems[b], add=True)
                    hsc[0].wait()
                    hsc[1].wait()

            plsc.subcore_barrier()

        with jax.named_scope("ph_out"):
            pltpu.sync_copy(aggsh.at[pl.ds(s * _RPT, _RPT)],
                            agg_out.at[c, pl.ds(s * _RPT, _RPT)])

            @pl.when(jnp.logical_and(c == 0, s < _NS - 1))
            def _():
                pltpu.sync_copy(dinvv.at[pl.ds(s * _RPT, _RPT)],
                                dinv_out.at[pl.ds(s * _RPT, _RPT)])

            @pl.when(jnp.logical_and(c == 0, s == _NS - 1))
            def _():
                pltpu.sync_copy(dinvv.at[pl.ds(s * _RPT, _N % _RPT)],
                                dinv_out.at[pl.ds(s * _RPT, _N % _RPT)])

    return k(x, e3, ew3)


def _tc_tail(agg_ref, x_ref, x0_ref, dinv_ref, w_ref, g_ref, b_ref, y_ref):
    dsq = dinv_ref[...] * dinv_ref[...]
    agg = agg_ref[0, :_N, :] + agg_ref[1, :_N, :] + x_ref[...] * dsq
    h = (1.0 - _ALPHA) * agg + _ALPHA * x0_ref[...]
    out = jnp.dot(h, w_ref[...], preferred_element_type=jnp.float32,
                  precision=lax.Precision.HIGHEST)
    out = jnp.maximum(out, 0.0)
    mean = jnp.sum(out, axis=0) / _N
    msq = jnp.sum(out * out, axis=0) / _N
    var = msq - mean * mean
    scale = g_ref[...] * lax.rsqrt(var + _EPS)[None, :]
    y_ref[...] = (out - mean[None, :]) * scale + b_ref[...]


def kernel(x, x_orig, edge_index, edge_weight, W, gamma, beta):
    e3 = edge_index.reshape(2, _NCH, _CH)
    ew3 = edge_weight.reshape(_NCH, _CH)

    aggp, dinv = _sc_gcn_agg(x, e3, ew3)

    y = pl.pallas_call(
        _tc_tail,
        out_shape=jax.ShapeDtypeStruct((_N, _D), jnp.float32),
    )(aggp, x, x_orig, dinv[:, None], W, gamma[None, :], beta[None, :])

    return (y, x_orig, edge_index, edge_weight, x)


# speedup vs baseline: 1.2463x; 1.2463x over previous
import dataclasses
import functools

import jax
import jax.numpy as jnp
from jax import lax
from jax.experimental import pallas as pl
from jax.experimental.pallas import tpu as pltpu
from jax.experimental.pallas import tpu_sc as plsc

_N = 10000
_E = 320000
_D = 128
_ALPHA = 0.1
_EPS = 1e-5

_NC = 2
_NS = 16
_L = 16
_NW = _NC * _NS

_CH = 80
_CB = 8
_NCH = _E // _CH
_NBT = _NCH // _CB
_BPT = 16
_BPTD = 32
_NPAD = 10240
_RPT = _NPAD // _NS


def _rsqrt16(d):
    i = plsc.bitcast(d, jnp.int32)
    i = jnp.int32(0x5F3759DF) - lax.shift_right_logical(i, 1)
    y = plsc.bitcast(i, jnp.float32)
    for _ in range(3):
        y = y * (1.5 - 0.5 * d * y * y)
    return y


def _sc_gcn_agg(x, e3, ew3):
    mesh = plsc.VectorSubcoreMesh(core_axis_name="c", subcore_axis_name="s")
    cp = pltpu.CompilerParams()
    if "needs_layout_passes" in pltpu.CompilerParams.__dataclass_fields__:
        cp = dataclasses.replace(cp, needs_layout_passes=False)
    if "use_tc_tiling_on_sc" in pltpu.CompilerParams.__dataclass_fields__:
        cp = dataclasses.replace(cp, use_tc_tiling_on_sc=False)

    @functools.partial(
        pl.kernel,
        compiler_params=cp,
        out_type=(
            jax.ShapeDtypeStruct((_NC, _NPAD, _D), jnp.float32),
            jax.ShapeDtypeStruct((_N,), jnp.float32),
        ),
        mesh=mesh,
        scratch_types=[
            pltpu.VMEM((_CB, _CH), jnp.int32),
            pltpu.VMEM((_CB, _CH), jnp.int32),
            pltpu.VMEM((_CB, _CH), jnp.float32),
            pltpu.VMEM((_N,), jnp.float32),
            pltpu.VMEM((2, _CH, _L), jnp.float32),
            pltpu.VMEM((2, _CH, _D), jnp.float32),
            pltpu.VMEM((_CH,), jnp.float32),
            pltpu.VMEM_SHARED((_NPAD, _L), jnp.float32),
            pltpu.VMEM_SHARED((_NPAD, _D), jnp.float32),
            pltpu.VMEM_SHARED((_N,), jnp.float32),
        ] + [pltpu.SemaphoreType.DMA] * 6,
    )
    def k(x_hbm, e3_hbm, ew3_hbm, agg_out, dinv_out,
          rowblk, colblk, ewblk, dinvv, valbuf, msgbuf, normbuf,
          degmat, aggsh, dinvsh,
          sg0, sg1, ss0, ss1, sd0, sd1):
        semg = [sg0, sg1]
        sems = [ss0, ss1]
        semd = [sd0, sd1]
        c = lax.axis_index("c")
        s = lax.axis_index("s")
        iota16 = lax.iota(jnp.int32, _L)
        zero16i = jnp.zeros((_L,), jnp.int32)
        z16 = jnp.zeros((_L,), jnp.float32)

        with jax.named_scope("ph_zero"):
            @pl.loop(0, _CH)
            def _(e):
                valbuf[0, e, :] = z16
                valbuf[1, e, :] = z16
                for g in range(_D // _L):
                    msgbuf[0, e, pl.ds(g * _L, _L)] = z16

            for i in range(_RPT // _CH):
                base = s * _RPT + i * _CH
                pltpu.sync_copy(valbuf.at[0], degmat.at[pl.ds(base, _CH)])
                pltpu.sync_copy(msgbuf.at[0], aggsh.at[pl.ds(base, _CH)])

            plsc.subcore_barrier()

        with jax.named_scope("ph_deg"):
            @pl.loop(0, _BPTD)
            def _(jb):
                gb = s * _BPTD + jb

                @pl.when(gb < _NBT)
                def _():
                    pltpu.sync_copy(e3_hbm.at[1, pl.ds(gb * _CB, _CB)],
                                    colblk)
                    pltpu.sync_copy(ew3_hbm.at[pl.ds(gb * _CB, _CB)], ewblk)
                    hs = {}
                    for j8 in range(_CB):
                        b = j8 % 2
                        if j8 >= 2:
                            hs[b].wait()
                        for g in range(_CH // _L):
                            ew16 = ewblk[j8, pl.ds(g * _L, _L)]
                            plsc.store_scatter(
                                valbuf.at[b], [g * _L + iota16, zero16i],
                                ew16)
                        hs[b] = pltpu.async_copy(
                            valbuf.at[b], degmat.at[colblk.at[j8]],
                            semd[b], add=True)
                    hs[0].wait()
                    hs[1].wait()

            plsc.subcore_barrier()

        with jax.named_scope("ph_rsqrt"):
            for seg in range(_RPT // _CH):
                base = s * _RPT + seg * _CH
                pltpu.sync_copy(degmat.at[pl.ds(base, _CH)], valbuf.at[0])

                @pl.loop(0, _CH // _L)
                def _(g, base=base):
                    gbase = base + g * _L

                    @pl.when(gbase < _N)
                    def _():
                        r16 = g * _L + iota16
                        d16 = plsc.load_gather(valbuf.at[0], [r16, zero16i])
                        dinvv[pl.ds(gbase, _L)] = _rsqrt16(d16 + 1.0)

            @pl.when(s < _NS - 1)
            def _():
                pltpu.sync_copy(dinvv.at[pl.ds(s * _RPT, _RPT)],
                                dinvsh.at[pl.ds(s * _RPT, _RPT)])

            @pl.when(s == _NS - 1)
            def _():
                pltpu.sync_copy(dinvv.at[pl.ds(s * _RPT, _N % _RPT)],
                                dinvsh.at[pl.ds(s * _RPT, _N % _RPT)])

            plsc.subcore_barrier()
            pltpu.sync_copy(dinvsh, dinvv)

        with jax.named_scope("ph_agg"):
            w = c * _NS + s

            @pl.loop(0, _BPT)
            def _(jb):
                gb = w * _BPT + jb

                @pl.when(gb < _NBT)
                def _():
                    pltpu.sync_copy(e3_hbm.at[0, pl.ds(gb * _CB, _CB)],
                                    rowblk)
                    pltpu.sync_copy(e3_hbm.at[1, pl.ds(gb * _CB, _CB)],
                                    colblk)
                    pltpu.sync_copy(ew3_hbm.at[pl.ds(gb * _CB, _CB)], ewblk)
                    hg, hsc = {}, {}
                    hg[0] = pltpu.async_copy(
                        x_hbm.at[rowblk.at[0]], msgbuf.at[0], semg[0])
                    for j8 in range(_CB):
                        b = j8 % 2
                        hg[b].wait()
                        if j8 + 1 < _CB:
                            b1 = (j8 + 1) % 2
                            if j8 >= 1:
                                hsc[b1].wait()
                            hg[b1] = pltpu.async_copy(
                                x_hbm.at[rowblk.at[j8 + 1]], msgbuf.at[b1],
                                semg[b1])
                        for g in range(_CH // _L):
                            sl = pl.ds(g * _L, _L)
                            r16 = rowblk[j8, sl]
                            c16 = colblk[j8, sl]
                            ew16 = ewblk[j8, sl]
                            dr = plsc.load_gather(dinvv, [r16])
                            dc = plsc.load_gather(dinvv, [c16])
                            normbuf[sl] = dr * ew16 * dc

                        @pl.loop(0, _CH // _L)
                        def _(eo, b=b):
                            n16 = normbuf[pl.ds(eo * _L, _L)]
                            for kk in range(_L):
                                ne = n16[kk]
                                for gg in range(_D // _L):
                                    sld = pl.ds(gg * _L, _L)
                                    msgbuf[b, eo * _L + kk, sld] = \
                                        msgbuf[b, eo * _L + kk, sld] * ne

                        hsc[b] = pltpu.async_copy(
                            msgbuf.at[b], aggsh.at[colblk.at[j8]],
                            sems[b], add=True)
                    hsc[0].wait()
                    hsc[1].wait()

            plsc.subcore_barrier()

        with jax.named_scope("ph_out"):
            pltpu.sync_copy(aggsh.at[pl.ds(s * _RPT, _RPT)],
                            agg_out.at[c, pl.ds(s * _RPT, _RPT)])

            @pl.when(jnp.logical_and(c == 0, s < _NS - 1))
            def _():
                pltpu.sync_copy(dinvv.at[pl.ds(s * _RPT, _RPT)],
                                dinv_out.at[pl.ds(s * _RPT, _RPT)])

            @pl.when(jnp.logical_and(c == 0, s == _NS - 1))
            def _():
                pltpu.sync_copy(dinvv.at[pl.ds(s * _RPT, _N % _RPT)],
                                dinv_out.at[pl.ds(s * _RPT, _N % _RPT)])

    return k(x, e3, ew3)


def _tc_tail(agg_ref, x_ref, x0_ref, dinv_ref, w_ref, g_ref, b_ref, y_ref):
    dsq = dinv_ref[...] * dinv_ref[...]
    agg = agg_ref[0, :_N, :] + agg_ref[1, :_N, :] + x_ref[...] * dsq
    h = (1.0 - _ALPHA) * agg + _ALPHA * x0_ref[...]
    out = jnp.dot(h, w_ref[...], preferred_element_type=jnp.float32,
                  precision=lax.Precision.HIGHEST)
    out = jnp.maximum(out, 0.0)
    mean = jnp.sum(out, axis=0) / _N
    msq = jnp.sum(out * out, axis=0) / _N
    var = msq - mean * mean
    scale = g_ref[...] * lax.rsqrt(var + _EPS)[None, :]
    y_ref[...] = (out - mean[None, :]) * scale + b_ref[...]


def kernel(x, x_orig, edge_index, edge_weight, W, gamma, beta):
    e3 = edge_index.reshape(2, _NCH, _CH)
    ew3 = edge_weight.reshape(_NCH, _CH)

    aggp, dinv = _sc_gcn_agg(x, e3, ew3)

    y = pl.pallas_call(
        _tc_tail,
        out_shape=jax.ShapeDtypeStruct((_N, _D), jnp.float32),
    )(aggp, x, x_orig, dinv[:, None], W, gamma[None, :], beta[None, :])

    return (y, x_orig, edge_index, edge_weight, x)


# speedup vs baseline: 1.3350x; 1.0712x over previous
import dataclasses
import functools

import jax
import jax.numpy as jnp
from jax import lax
from jax.experimental import pallas as pl
from jax.experimental.pallas import tpu as pltpu
from jax.experimental.pallas import tpu_sc as plsc

_N = 10000
_E = 320000
_D = 128
_ALPHA = 0.1
_EPS = 1e-5

_NC = 2
_NS = 16
_L = 16
_NW = _NC * _NS

_CH = 80
_CB = 8
_NCH = _E // _CH
_NBT = _NCH // _CB
_BPT = 16
_BPTD = 32
_NPAD = 10240
_RPT = _NPAD // _NS


def _rsqrt16(d):
    i = plsc.bitcast(d, jnp.int32)
    i = jnp.int32(0x5F3759DF) - lax.shift_right_logical(i, 1)
    y = plsc.bitcast(i, jnp.float32)
    for _ in range(3):
        y = y * (1.5 - 0.5 * d * y * y)
    return y


def _sc_gcn_agg(x, e3, ew3):
    mesh = plsc.VectorSubcoreMesh(core_axis_name="c", subcore_axis_name="s")
    cp = pltpu.CompilerParams()
    if "needs_layout_passes" in pltpu.CompilerParams.__dataclass_fields__:
        cp = dataclasses.replace(cp, needs_layout_passes=False)
    if "use_tc_tiling_on_sc" in pltpu.CompilerParams.__dataclass_fields__:
        cp = dataclasses.replace(cp, use_tc_tiling_on_sc=False)

    @functools.partial(
        pl.kernel,
        compiler_params=cp,
        out_type=(
            jax.ShapeDtypeStruct((_NC, _NPAD, _D), jnp.float32),
            jax.ShapeDtypeStruct((_N,), jnp.float32),
        ),
        mesh=mesh,
        scratch_types=[
            pltpu.VMEM((2, _CB, _CH), jnp.int32),
            pltpu.VMEM((2, _CB, _CH), jnp.int32),
            pltpu.VMEM((2, _CB, _CH), jnp.float32),
            pltpu.VMEM((_N,), jnp.float32),
            pltpu.VMEM((2, _CH, _L), jnp.float32),
            pltpu.VMEM((2, _CH, _D), jnp.float32),
            pltpu.VMEM((_CH,), jnp.float32),
            pltpu.VMEM_SHARED((_NPAD, _L), jnp.float32),
            pltpu.VMEM_SHARED((_NPAD, _D), jnp.float32),
            pltpu.VMEM_SHARED((_N,), jnp.float32),
        ] + [pltpu.SemaphoreType.DMA] * 8,
    )
    def k(x_hbm, e3_hbm, ew3_hbm, agg_out, dinv_out,
          rowblk, colblk, ewblk, dinvv, valbuf, msgbuf, normbuf,
          degmat, aggsh, dinvsh,
          sg0, sg1, ss0, ss1, sd0, sd1, sp0, sp1):
        semg = [sg0, sg1]
        sems = [ss0, ss1]
        semd = [sd0, sd1]
        semp = [sp0, sp1]
        c = lax.axis_index("c")
        s = lax.axis_index("s")
        iota16 = lax.iota(jnp.int32, _L)
        zero16i = jnp.zeros((_L,), jnp.int32)
        z16 = jnp.zeros((_L,), jnp.float32)

        with jax.named_scope("ph_zero"):
            @pl.loop(0, _CH)
            def _(e):
                valbuf[0, e, :] = z16
                valbuf[1, e, :] = z16
                for g in range(_D // _L):
                    msgbuf[0, e, pl.ds(g * _L, _L)] = z16

            for i in range(_RPT // _CH):
                base = s * _RPT + i * _CH
                pltpu.sync_copy(valbuf.at[0], degmat.at[pl.ds(base, _CH)])
                pltpu.sync_copy(msgbuf.at[0], aggsh.at[pl.ds(base, _CH)])

            plsc.subcore_barrier()

        with jax.named_scope("ph_deg"):
            @pl.loop(0, _BPTD)
            def _(jb):
                gb = s * _BPTD + jb

                @pl.when(gb < _NBT)
                def _():
                    pltpu.sync_copy(e3_hbm.at[1, pl.ds(gb * _CB, _CB)],
                                    colblk.at[0])
                    pltpu.sync_copy(ew3_hbm.at[pl.ds(gb * _CB, _CB)],
                                    ewblk.at[0])
                    hs = {}
                    for j8 in range(_CB):
                        b = j8 % 2
                        if j8 >= 2:
                            hs[b].wait()
                        for g in range(_CH // _L):
                            ew16 = ewblk[0, j8, pl.ds(g * _L, _L)]
                            plsc.store_scatter(
                                valbuf.at[b], [g * _L + iota16, zero16i],
                                ew16)
                        hs[b] = pltpu.async_copy(
                            valbuf.at[b], degmat.at[colblk.at[0, j8]],
                            semd[b], add=True)
                    hs[0].wait()
                    hs[1].wait()

            plsc.subcore_barrier()

        with jax.named_scope("ph_rsqrt"):
            for seg in range(_RPT // _CH):
                base = s * _RPT + seg * _CH
                pltpu.sync_copy(degmat.at[pl.ds(base, _CH)], valbuf.at[0])

                @pl.loop(0, _CH // _L)
                def _(g, base=base):
                    gbase = base + g * _L

                    @pl.when(gbase < _N)
                    def _():
                        r16 = g * _L + iota16
                        d16 = plsc.load_gather(valbuf.at[0], [r16, zero16i])
                        dinvv[pl.ds(gbase, _L)] = _rsqrt16(d16 + 1.0)

            @pl.when(s < _NS - 1)
            def _():
                pltpu.sync_copy(dinvv.at[pl.ds(s * _RPT, _RPT)],
                                dinvsh.at[pl.ds(s * _RPT, _RPT)])

            @pl.when(s == _NS - 1)
            def _():
                pltpu.sync_copy(dinvv.at[pl.ds(s * _RPT, _N % _RPT)],
                                dinvsh.at[pl.ds(s * _RPT, _N % _RPT)])

            plsc.subcore_barrier()
            pltpu.sync_copy(dinvsh, dinvv)

        with jax.named_scope("ph_agg"):
            w = c * _NS + s
            w0 = w * _BPT

            def prefetch(gb, slot):
                pltpu.async_copy(e3_hbm.at[0, pl.ds(gb * _CB, _CB)],
                                 rowblk.at[slot], semp[slot])
                pltpu.async_copy(e3_hbm.at[1, pl.ds(gb * _CB, _CB)],
                                 colblk.at[slot], semp[slot])
                pltpu.async_copy(ew3_hbm.at[pl.ds(gb * _CB, _CB)],
                                 ewblk.at[slot], semp[slot])

            def wait_prefetch(slot):
                pltpu.make_async_copy(e3_hbm.at[0, pl.ds(0, _CB)],
                                      rowblk.at[slot], semp[slot]).wait()
                pltpu.make_async_copy(e3_hbm.at[1, pl.ds(0, _CB)],
                                      colblk.at[slot], semp[slot]).wait()
                pltpu.make_async_copy(ew3_hbm.at[pl.ds(0, _CB)],
                                      ewblk.at[slot], semp[slot]).wait()

            def process_block(slot):
                hg, hsc = {}, {}
                hg[0] = pltpu.async_copy(
                    x_hbm.at[rowblk.at[slot, 0]], msgbuf.at[0], semg[0])
                for j8 in range(_CB):
                    b = j8 % 2
                    hg[b].wait()
                    if j8 + 1 < _CB:
                        b1 = (j8 + 1) % 2
                        if j8 >= 1:
                            hsc[b1].wait()
                        hg[b1] = pltpu.async_copy(
                            x_hbm.at[rowblk.at[slot, j8 + 1]], msgbuf.at[b1],
                            semg[b1])
                    for g in range(_CH // _L):
                        sl = pl.ds(g * _L, _L)
                        r16 = rowblk[slot, j8, sl]
                        c16 = colblk[slot, j8, sl]
                        ew16 = ewblk[slot, j8, sl]
                        dr = plsc.load_gather(dinvv, [r16])
                        dc = plsc.load_gather(dinvv, [c16])
                        normbuf[sl] = dr * ew16 * dc

                    @pl.loop(0, _CH // _L)
                    def _(eo, b=b):
                        n16 = normbuf[pl.ds(eo * _L, _L)]
                        for kk in range(_L):
                            ne = n16[kk]
                            for gg in range(_D // _L):
                                sld = pl.ds(gg * _L, _L)
                                msgbuf[b, eo * _L + kk, sld] = \
                                    msgbuf[b, eo * _L + kk, sld] * ne

                    hsc[b] = pltpu.async_copy(
                        msgbuf.at[b], aggsh.at[colblk.at[slot, j8]],
                        sems[b], add=True)
                hsc[0].wait()
                hsc[1].wait()

            @pl.when(w0 < _NBT)
            def _():
                prefetch(w0, 0)

            @pl.loop(0, _BPT // 2)
            def _(m):
                b0 = w0 + 2 * m
                b1g = b0 + 1
                b2g = b0 + 2

                @pl.when(b0 < _NBT)
                def _():
                    wait_prefetch(0)

                    @pl.when(b1g < _NBT)
                    def _():
                        prefetch(b1g, 1)

                    process_block(0)

                    @pl.when(jnp.logical_and(b2g < _NBT,
                                             2 * m + 2 < _BPT))
                    def _():
                        prefetch(b2g, 0)

                @pl.when(b1g < _NBT)
                def _():
                    wait_prefetch(1)
                    process_block(1)

            plsc.subcore_barrier()

        with jax.named_scope("ph_out"):
            pltpu.sync_copy(aggsh.at[pl.ds(s * _RPT, _RPT)],
                            agg_out.at[c, pl.ds(s * _RPT, _RPT)])

            @pl.when(jnp.logical_and(c == 0, s < _NS - 1))
            def _():
                pltpu.sync_copy(dinvv.at[pl.ds(s * _RPT, _RPT)],
                                dinv_out.at[pl.ds(s * _RPT, _RPT)])

            @pl.when(jnp.logical_and(c == 0, s == _NS - 1))
            def _():
                pltpu.sync_copy(dinvv.at[pl.ds(s * _RPT, _N % _RPT)],
                                dinv_out.at[pl.ds(s * _RPT, _N % _RPT)])

    return k(x, e3, ew3)


def _tc_tail(agg_ref, x_ref, x0_ref, dinv_ref, w_ref, g_ref, b_ref, y_ref):
    dsq = dinv_ref[...] * dinv_ref[...]
    agg = agg_ref[0, :_N, :] + agg_ref[1, :_N, :] + x_ref[...] * dsq
    h = (1.0 - _ALPHA) * agg + _ALPHA * x0_ref[...]
    out = jnp.dot(h, w_ref[...], preferred_element_type=jnp.float32,
                  precision=lax.Precision.HIGHEST)
    out = jnp.maximum(out, 0.0)
    mean = jnp.sum(out, axis=0) / _N
    msq = jnp.sum(out * out, axis=0) / _N
    var = msq - mean * mean
    scale = g_ref[...] * lax.rsqrt(var + _EPS)[None, :]
    y_ref[...] = (out - mean[None, :]) * scale + b_ref[...]


def kernel(x, x_orig, edge_index, edge_weight, W, gamma, beta):
    e3 = edge_index.reshape(2, _NCH, _CH)
    ew3 = edge_weight.reshape(_NCH, _CH)

    aggp, dinv = _sc_gcn_agg(x, e3, ew3)

    y = pl.pallas_call(
        _tc_tail,
        out_shape=jax.ShapeDtypeStruct((_N, _D), jnp.float32),
    )(aggp, x, x_orig, dinv[:, None], W, gamma[None, :], beta[None, :])

    return (y, x_orig, edge_index, edge_weight, x)
